# baseline (speedup 1.0000x reference)
import functools

import jax
import jax.numpy as jnp
from jax import lax
from jax.experimental import pallas as pl
from jax.experimental.pallas import tpu as pltpu
from jax.experimental.pallas import tpu_sc as plsc

_D = 32
_NW = 32
_B = 4096
_H = 200
_BLK = _B // _NW


@functools.cache
def _make_gather():
    n_rows_out = _H * (_D // 8) * _NW * 8
    per_w = _BLK * _H
    mesh = plsc.VectorSubcoreMesh(core_axis_name="c", subcore_axis_name="s")

    @functools.partial(
        pl.kernel,
        mesh=mesh,
        out_type=jax.ShapeDtypeStruct((n_rows_out, 128), jnp.float32),
        scratch_types=[
            pltpu.VMEM((per_w,), jnp.int32),
            pltpu.VMEM((_BLK,), jnp.int32),
            pltpu.VMEM((_BLK,), jnp.int32),
            pltpu.VMEM((_BLK, _D), jnp.float32),
            pltpu.VMEM((_BLK, _D), jnp.float32),
            pltpu.VMEM((_D, _BLK), jnp.float32),
            pltpu.VMEM((_D, _BLK), jnp.float32),
            pltpu.SemaphoreType.DMA,
            pltpu.SemaphoreType.DMA,
            pltpu.SemaphoreType.DMA,
            pltpu.SemaphoreType.DMA,
        ],
        compiler_params=pltpu.CompilerParams(
            use_tc_tiling_on_sc=False, needs_layout_passes=False
        ),
    )
    def body(tokens_hbm, table_hbm, out_hbm, tok_v, i0, i1, r0, r1, t0, t1,
             g0, g1, o0, o1):
        idx = (i0, i1)
        rows = (r0, r1)
        tbuf = (t0, t1)
        gsem = (g0, g1)
        osem = (o0, o1)
        blk = lax.axis_index("s") * 2 + lax.axis_index("c")
        iota = lax.iota(jnp.int32, 16)
        iota_h = iota * _H
        iota_r = iota

        pltpu.sync_copy(tokens_hbm.at[pl.ds(blk * per_w, per_w)], tok_v)

        def fire(h, b):
            for g in range(_BLK // 16):
                vec = iota_h + (g * 16 * _H + h)
                idx[b][pl.ds(g * 16, 16)] = plsc.load_gather(tok_v, [vec])
            pltpu.async_copy(table_hbm.at[idx[b]], rows[b], gsem[b])

        def out_slices(h, b):
            r0_ = h * (_NW * _D) + blk * 8
            return [
                (tbuf[b].at[pl.ds(k * 8, 8), :],
                 out_hbm.at[pl.ds(r0_ + k * 256, 8), :])
                for k in range(_D // 8)
            ]

        def process(h, b):
            pltpu.make_async_copy(table_hbm.at[idx[b]], rows[b], gsem[b]).wait()
            for e in range(_D):
                col = jnp.full((16,), e, jnp.int32)
                for g in range(_BLK // 16):
                    tbuf[b][e, pl.ds(g * 16, 16)] = plsc.load_gather(
                        rows[b], [iota_r + g * 16, col]
                    )
            for src, dst in out_slices(h, b):
                pltpu.async_copy(src, dst, osem[b])

        def wait_out(h, b):
            for src, dst in out_slices(h, b):
                pltpu.make_async_copy(src, dst, osem[b]).wait()

        fire(0, 0)

        def pair(cj, carry):
            i = 2 * cj
            fire(i + 1, 1)

            @pl.when(cj > 0)
            def _():
                wait_out(i - 2, 0)

            process(i, 0)

            @pl.when(cj < _H // 2 - 1)
            def _():
                fire(i + 2, 0)

            @pl.when(cj > 0)
            def _():
                wait_out(i - 1, 1)

            process(i + 1, 1)
            return carry

        lax.fori_loop(0, _H // 2, pair, 0)
        wait_out(_H - 2, 0)
        wait_out(_H - 1, 1)

    return body


def kernel(tokens, table):
    b, h = tokens.shape
    d = table.shape[1]
    flat = tokens.reshape(-1).astype(jnp.int32)
    out2d = _make_gather()(flat, table)
    x = out2d.reshape(h, d // 8, _NW, 8, 128)
    x = x.transpose(2, 4, 0, 1, 3)
    return x.reshape(b, h, d)


# speedup vs baseline: 1.6274x; 1.6274x over previous
import functools

import jax
import jax.numpy as jnp
from jax import lax
from jax.experimental import pallas as pl
from jax.experimental.pallas import tpu as pltpu
from jax.experimental.pallas import tpu_sc as plsc

_D = 32
_NW = 32
_CHUNK = 512
_GRP = 128


@functools.cache
def _make_gather(n_rows: int, d: int):
    per_w = n_rows // _NW
    n_chunks = per_w // _CHUNK
    mesh = plsc.VectorSubcoreMesh(core_axis_name="c", subcore_axis_name="s")

    @functools.partial(
        pl.kernel,
        mesh=mesh,
        out_type=jax.ShapeDtypeStruct((n_rows, 128), jnp.float32),
        scratch_types=[
            pltpu.VMEM((_CHUNK,), jnp.int32),
            pltpu.VMEM((_CHUNK,), jnp.int32),
            pltpu.VMEM((_CHUNK, d), jnp.float32),
            pltpu.VMEM((_CHUNK, d), jnp.float32),
            pltpu.SemaphoreType.DMA,
            pltpu.SemaphoreType.DMA,
            pltpu.SemaphoreType.DMA,
            pltpu.SemaphoreType.DMA,
        ],
        compiler_params=pltpu.CompilerParams(use_tc_tiling_on_sc=False),
    )
    def body(tokens_hbm, table_hbm, out_hbm, idx_v, idx_b, rows_v, rows_b, sem, semb, osem, osemb):
        wid = lax.axis_index("s") * 2 + lax.axis_index("c")
        base = wid * per_w
        idx = (idx_v, idx_b)
        rows = (rows_v, rows_b)
        gsem = (sem, semb)
        wsem = (osem, osemb)
        n_pairs = n_chunks // 2

        def fire(ci, b):
            off = base + ci * _CHUNK
            pltpu.sync_copy(tokens_hbm.at[pl.ds(off, _CHUNK)], idx[b])
            for g in range(_CHUNK // _GRP):
                pltpu.async_copy(
                    table_hbm.at[idx[b].at[pl.ds(g * _GRP, _GRP)]],
                    rows[b].at[pl.ds(g * _GRP, _GRP)],
                    gsem[b],
                )

        def drain_and_write(ci, b):
            for g in range(_CHUNK // _GRP):
                pltpu.make_async_copy(
                    table_hbm.at[idx[b].at[pl.ds(g * _GRP, _GRP)]],
                    rows[b].at[pl.ds(g * _GRP, _GRP)],
                    gsem[b],
                ).wait()
            off = base + ci * _CHUNK
            pltpu.async_copy(
                rows[b], out_hbm.at[pl.ds(off, _CHUNK), pl.ds(0, d)], wsem[b]
            )

        def wait_write(ci, b):
            off = base + ci * _CHUNK
            pltpu.make_async_copy(
                rows[b], out_hbm.at[pl.ds(off, _CHUNK), pl.ds(0, d)], wsem[b]
            ).wait()

        fire(0, 0)

        def pair(cj, carry):
            i = 2 * cj + 1

            @pl.when(cj > 0)
            def _():
                wait_write(i - 2, 1)

            fire(i, 1)
            drain_and_write(i - 1, 0)

            @pl.when(cj < n_pairs - 1)
            def _():
                wait_write(i - 1, 0)
                fire(i + 1, 0)

            drain_and_write(i, 1)
            return carry

        lax.fori_loop(0, n_pairs, pair, 0)
        wait_write(n_chunks - 2, 0)
        wait_write(n_chunks - 1, 1)

    return body


def kernel(tokens, table):
    b, h = tokens.shape
    d = table.shape[1]
    flat = tokens.reshape(-1).astype(jnp.int32)
    out = _make_gather(flat.shape[0], d)(flat, table)
    return out[:, :d].reshape(b, h, d)
